# baseline (speedup 1.0000x reference)
import functools

import jax
import jax.numpy as jnp
from jax.experimental import pallas as pl
from jax.experimental.pallas import tpu as pltpu

_LANE = 128
_SUBLANE = 8
_BR = 32


def _round_up(x, m):
    return ((x + m - 1) // m) * m


def _loss_kernel(idx_ref, tgt_ref, tab16_ref, m_ref, logits_ref, part_ref,
                 *, br, t, n_rows):
    pid = pl.program_id(0)
    v_pad = tab16_ref.shape[0]
    rowv = jax.lax.broadcasted_iota(jnp.int32, (v_pad, t), 0)

    counts = jnp.zeros((v_pad, v_pad), jnp.float32)
    for j in range(br):
        ids_j = idx_ref[j:j + 1, :]
        tgt_j = tgt_ref[j:j + 1, :]
        oh_i = (rowv == ids_j).astype(jnp.bfloat16)
        valid = rowv == tgt_j
        if n_rows % br != 0:
            valid = valid & (pid * br + j < n_rows)
        oh_t = valid.astype(jnp.bfloat16)

        logits_ref[pl.ds(j * t, t), :] = jax.lax.dot_general(
            oh_i, tab16_ref[...], (((0,), (0,)), ((), ())),
            preferred_element_type=jnp.float32)

        counts = counts + jax.lax.dot_general(
            oh_i, oh_t, (((1,), (1,)), ((), ())),
            preferred_element_type=jnp.float32)

    part_ref[...] = jnp.broadcast_to(
        jnp.sum(counts * m_ref[...]), part_ref.shape)


@functools.partial(jax.jit, static_argnames=("vocab",))
def _bigram_loss(idx, targets, table, *, vocab):
    B, T = idx.shape
    n = B * T
    v_pad = _round_up(vocab, _LANE)
    br = _BR
    b_pad = _round_up(B, br)
    num_blocks = b_pad // br

    idx_p = idx if b_pad == B else jnp.pad(idx, ((0, b_pad - B), (0, 0)))
    tgt_p = targets if b_pad == B else jnp.pad(targets, ((0, b_pad - B), (0, 0)))
    table16 = jnp.pad(table, ((0, v_pad - vocab), (0, v_pad - vocab))
                      ).astype(jnp.bfloat16)

    row_lse = jax.scipy.special.logsumexp(table, axis=-1)
    m_mat = jnp.pad(row_lse[:, None] - table,
                    ((0, v_pad - vocab), (0, v_pad - vocab)))

    body = functools.partial(_loss_kernel, br=br, t=T, n_rows=B)
    logits_p, parts = pl.pallas_call(
        body,
        out_shape=(
            jax.ShapeDtypeStruct((b_pad * T, v_pad), jnp.float32),
            jax.ShapeDtypeStruct((num_blocks, _SUBLANE, _LANE), jnp.float32),
        ),
        grid_spec=pltpu.PrefetchScalarGridSpec(
            num_scalar_prefetch=0,
            grid=(num_blocks,),
            in_specs=[
                pl.BlockSpec((br, T), lambda i: (i, 0)),
                pl.BlockSpec((br, T), lambda i: (i, 0)),
                pl.BlockSpec((v_pad, v_pad), lambda i: (0, 0)),
                pl.BlockSpec((v_pad, v_pad), lambda i: (0, 0)),
            ],
            out_specs=(
                pl.BlockSpec((br * T, v_pad), lambda i: (i, 0)),
                pl.BlockSpec((1, _SUBLANE, _LANE), lambda i: (i, 0, 0)),
            ),
        ),
        compiler_params=pltpu.CompilerParams(
            dimension_semantics=("parallel",),
            vmem_limit_bytes=100 * 1024 * 1024,
        ),
    )(idx_p, tgt_p, table16, m_mat)

    loss = jnp.sum(parts[:, 0, 0]) / n
    return logits_p[:n, :vocab], loss


def kernel(idx, targets, token_embedding_table):
    vocab = token_embedding_table.shape[0]
    B, T = idx.shape
    idx = idx.astype(jnp.int32)
    if targets is None:
        logits, _ = _bigram_loss(idx, jnp.zeros_like(idx),
                                 token_embedding_table, vocab=vocab)
        return logits.reshape(B, T, vocab), None
    logits, loss = _bigram_loss(idx, targets.astype(jnp.int32),
                                token_embedding_table, vocab=vocab)
    return logits, loss


# speedup vs baseline: 1.0029x; 1.0029x over previous
import functools

import jax
import jax.numpy as jnp
from jax.experimental import pallas as pl
from jax.experimental.pallas import tpu as pltpu

_LANE = 128
_SUBLANE = 8
_BR = 32


def _round_up(x, m):
    return ((x + m - 1) // m) * m


def _loss_kernel(idx_ref, tgt_ref, tab16_ref, m_ref, logits_ref, part_ref,
                 *, br, t, n_rows):
    pid = pl.program_id(0)
    v_pad = tab16_ref.shape[0]
    rowv = jax.lax.broadcasted_iota(jnp.int32, (v_pad, t), 0)

    counts = jnp.zeros((v_pad, v_pad), jnp.float32)
    for j in range(br):
        ids_j = idx_ref[j:j + 1, :]
        tgt_j = tgt_ref[j:j + 1, :]
        oh_i = (rowv == ids_j).astype(jnp.bfloat16)
        valid = rowv == tgt_j
        if n_rows % br != 0:
            valid = valid & (pid * br + j < n_rows)
        oh_t = valid.astype(jnp.bfloat16)

        logits_ref[pl.ds(j * t, t), :] = jax.lax.dot_general(
            oh_i, tab16_ref[...], (((0,), (0,)), ((), ())),
            preferred_element_type=jnp.float32)

        counts = counts + jax.lax.dot_general(
            oh_i, oh_t, (((1,), (1,)), ((), ())),
            preferred_element_type=jnp.float32)

    part_ref[...] = jnp.broadcast_to(
        jnp.sum(counts * m_ref[...]), part_ref.shape)


@functools.partial(jax.jit, static_argnames=("vocab",))
def _bigram_loss(idx, targets, table, *, vocab):
    B, T = idx.shape
    n = B * T
    v_pad = _round_up(vocab, _LANE)
    br = _BR
    b_pad = _round_up(B, br)
    num_blocks = b_pad // br

    idx_p = idx if b_pad == B else jnp.pad(idx, ((0, b_pad - B), (0, 0)))
    tgt_p = targets if b_pad == B else jnp.pad(targets, ((0, b_pad - B), (0, 0)))
    table16 = jnp.pad(table, ((0, v_pad - vocab), (0, v_pad - vocab))
                      ).astype(jnp.bfloat16)

    row_lse = jax.scipy.special.logsumexp(table, axis=-1)
    m_mat = jnp.pad(row_lse[:, None] - table,
                    ((0, v_pad - vocab), (0, v_pad - vocab)))

    body = functools.partial(_loss_kernel, br=br, t=T, n_rows=B)
    logits_p, parts = pl.pallas_call(
        body,
        out_shape=(
            jax.ShapeDtypeStruct((b_pad * T, v_pad), jnp.float32),
            jax.ShapeDtypeStruct((num_blocks, _SUBLANE, _LANE), jnp.float32),
        ),
        grid_spec=pltpu.PrefetchScalarGridSpec(
            num_scalar_prefetch=0,
            grid=(num_blocks,),
            in_specs=[
                pl.BlockSpec((br, T), lambda i: (i, 0)),
                pl.BlockSpec((br, T), lambda i: (i, 0)),
                pl.BlockSpec((v_pad, v_pad), lambda i: (0, 0)),
                pl.BlockSpec((v_pad, v_pad), lambda i: (0, 0)),
            ],
            out_specs=(
                pl.BlockSpec((br * T, v_pad), lambda i: (i, 0)),
                pl.BlockSpec((1, _SUBLANE, _LANE), lambda i: (i, 0, 0)),
            ),
        ),
        compiler_params=pltpu.CompilerParams(
            dimension_semantics=("parallel",),
            vmem_limit_bytes=48 * 1024 * 1024,
        ),
    )(idx_p, tgt_p, table16, m_mat)

    loss = jnp.sum(parts[:, 0, 0]) / n
    return logits_p[:n, :vocab], loss


def kernel(idx, targets, token_embedding_table):
    vocab = token_embedding_table.shape[0]
    B, T = idx.shape
    idx = idx.astype(jnp.int32)
    if targets is None:
        logits, _ = _bigram_loss(idx, jnp.zeros_like(idx),
                                 token_embedding_table, vocab=vocab)
        return logits.reshape(B, T, vocab), None
    logits, loss = _bigram_loss(idx, targets.astype(jnp.int32),
                                token_embedding_table, vocab=vocab)
    return logits, loss
